# baseline (speedup 1.0000x reference)
import functools

import jax
import jax.numpy as jnp
from jax import lax
from jax.experimental import pallas as pl
from jax.experimental.pallas import tpu as pltpu
from jax.experimental.pallas import tpu_sc as plsc

_NUM_EXPERT = 8
_IN = 1024
_OUT = 1024
_TOKENS = 2048
_BT = 256
_NB = _TOKENS // _BT
_NCHUNK = 2
_CROWS = _TOKENS // _NCHUNK
_CNB = _NB // _NCHUNK
_CNW = _CNB + _NUM_EXPERT - 1


def _sc_mesh():
    return plsc.VectorSubcoreMesh(core_axis_name="c", subcore_axis_name="s")


def _sc_gather_rows(src, idx):
    nrows = idx.shape[0]
    feat = src.shape[1]
    mesh = _sc_mesh()
    nworker = mesh.num_cores * mesh.num_subcores
    per_w = nrows // nworker

    @functools.partial(
        pl.kernel,
        mesh=mesh,
        out_type=jax.ShapeDtypeStruct((nrows, feat), src.dtype),
        scratch_types=[
            pltpu.VMEM((per_w,), jnp.int32),
            pltpu.VMEM((per_w, feat), src.dtype),
            pltpu.SemaphoreType.DMA,
        ],
    )
    def k(src_hbm, idx_hbm, out_hbm, idx_v, rows_v, sem):
        wid = lax.axis_index("s") * mesh.num_cores + lax.axis_index("c")
        base = wid * per_w
        pltpu.sync_copy(idx_hbm.at[pl.ds(base, per_w)], idx_v)
        pltpu.async_copy(src_hbm.at[idx_v], rows_v, sem).wait()
        pltpu.sync_copy(rows_v, out_hbm.at[pl.ds(base, per_w)])

    return k(src, idx)


def _sc_scatter_rows(ya, yb, idx):
    half, feat = ya.shape
    nrows = idx.shape[0]
    mesh = _sc_mesh()
    nworker = mesh.num_cores * mesh.num_subcores
    per_w = nrows // nworker
    w_split = half // per_w

    @functools.partial(
        pl.kernel,
        mesh=mesh,
        out_type=jax.ShapeDtypeStruct((nrows, feat), ya.dtype),
        scratch_types=[
            pltpu.VMEM((per_w,), jnp.int32),
            pltpu.VMEM((per_w, feat), ya.dtype),
            pltpu.SemaphoreType.DMA,
        ],
    )
    def k(ya_hbm, yb_hbm, idx_hbm, out_hbm, idx_v, rows_v, sem):
        wid = lax.axis_index("s") * mesh.num_cores + lax.axis_index("c")
        base = wid * per_w
        pltpu.sync_copy(idx_hbm.at[pl.ds(base, per_w)], idx_v)

        @pl.when(wid < w_split)
        def _():
            pltpu.sync_copy(ya_hbm.at[pl.ds(base, per_w)], rows_v)

        @pl.when(wid >= w_split)
        def _():
            pltpu.sync_copy(yb_hbm.at[pl.ds(base - half, per_w)], rows_v)

        pltpu.async_copy(rows_v, out_hbm.at[idx_v], sem).wait()

    return k(ya, yb, idx)


def _mm_body(meta_ref, x_ref, w_ref, g_ref, o_ref):
    w = pl.program_id(0)
    e_mask = meta_ref[2, w]
    first = meta_ref[3, w]
    xm = jnp.where(g_ref[...] == e_mask, x_ref[...], 0.0)
    part = lax.dot_general(
        xm, w_ref[0],
        dimension_numbers=(((1,), (1,)), ((), ())),
        preferred_element_type=jnp.float32,
    )

    @pl.when(first == 1)
    def _():
        o_ref[...] = part

    @pl.when(first == 0)
    def _():
        o_ref[...] += part


def _grouped_matmul(x_sorted, weight, sorted_gate, meta):
    rows = x_sorted.shape[0]
    grid_spec = pltpu.PrefetchScalarGridSpec(
        num_scalar_prefetch=1,
        grid=(meta.shape[1],),
        in_specs=[
            pl.BlockSpec((_BT, _IN), lambda w, m: (m[0, w], 0)),
            pl.BlockSpec((1, _OUT, _IN), lambda w, m: (m[1, w], 0, 0)),
            pl.BlockSpec((_BT, 1), lambda w, m: (m[0, w], 0)),
        ],
        out_specs=pl.BlockSpec((_BT, _OUT), lambda w, m: (m[0, w], 0)),
    )
    return pl.pallas_call(
        _mm_body,
        grid_spec=grid_spec,
        out_shape=jax.ShapeDtypeStruct((rows, _OUT), jnp.float32),
        compiler_params=pltpu.CompilerParams(
            dimension_semantics=("arbitrary",),
        ),
    )(meta, x_sorted, weight, sorted_gate.reshape(rows, 1))


def _routing(gate):
    g = gate.astype(jnp.int32)
    t = g.shape[0]
    eids = jnp.arange(_NUM_EXPERT, dtype=jnp.int32)
    oh3 = (g.reshape(128, t // 128, 1) == eids).astype(jnp.int32)
    c1 = jnp.cumsum(oh3, axis=1)
    row_tot = c1[:, -1, :]
    c2 = jnp.cumsum(row_tot, axis=0)
    excl = (c1 - oh3) + (c2 - row_tot)[:, None, :]
    pos = jnp.sum(oh3 * excl, axis=2).reshape(t)
    counts = c2[-1]
    off = jnp.cumsum(counts) - counts
    seg = jnp.sum(oh3 * off[None, None, :], axis=2).reshape(t)
    dest = (pos + seg).astype(jnp.int32)
    sort_idx = jnp.zeros(t, jnp.int32).at[dest].set(
        jnp.arange(t, dtype=jnp.int32))
    slot = jnp.arange(t, dtype=jnp.int32)
    sorted_gate = (jnp.sum((slot[:, None] >= off[None, :]).astype(jnp.int32),
                           axis=1) - 1).astype(jnp.int32)
    sgb = sorted_gate.reshape(_NB, _BT)
    e_lo_all, e_hi_all = sgb[:, 0], sgb[:, -1]
    metas = []
    warr = jnp.arange(_CNW, dtype=jnp.int32)
    for c in range(_NCHUNK):
        e_lo = e_lo_all[c * _CNB:(c + 1) * _CNB]
        e_hi = e_hi_all[c * _CNB:(c + 1) * _CNB]
        nitem = e_hi - e_lo + 1
        starts = jnp.cumsum(nitem) - nitem
        total = jnp.sum(nitem)
        b_of = jnp.sum((warr[:, None] >= starts[None, :]).astype(jnp.int32),
                       axis=1) - 1
        e_w = e_lo[b_of] + warr - starts[b_of]
        e_mask = jnp.where(warr < total, e_w, -1)
        e_load = jnp.clip(e_w, 0, _NUM_EXPERT - 1)
        firsts = (warr == starts[b_of]).astype(jnp.int32)
        metas.append(jnp.stack([b_of, e_load, e_mask, firsts]
                               ).astype(jnp.int32))
    return sort_idx, sorted_gate, metas


def kernel(inp, gate, weight):
    sort_idx, sorted_gate, metas = _routing(gate)
    ys = []
    for c in range(_NCHUNK):
        sl = slice(c * _CROWS, (c + 1) * _CROWS)
        x_c = _sc_gather_rows(inp, sort_idx[sl])
        ys.append(_grouped_matmul(x_c, weight, sorted_gate[sl], metas[c]))
    return _sc_scatter_rows(ys[0], ys[1], sort_idx)


# speedup vs baseline: 1.3444x; 1.3444x over previous
import functools

import jax
import jax.numpy as jnp
from jax import lax
from jax.experimental import pallas as pl
from jax.experimental.pallas import tpu as pltpu
from jax.experimental.pallas import tpu_sc as plsc

_NUM_EXPERT = 8
_IN = 1024
_OUT = 1024
_TOKENS = 2048
_BT = 256
_NB = _TOKENS // _BT
_NW = _NB + _NUM_EXPERT - 1


def _sc_permute(src, idx, scatter):
    rows, feat = src.shape
    mesh = plsc.VectorSubcoreMesh(core_axis_name="c", subcore_axis_name="s")
    nworker = mesh.num_cores * mesh.num_subcores
    per_w = rows // nworker

    @functools.partial(
        pl.kernel,
        mesh=mesh,
        out_type=jax.ShapeDtypeStruct((rows, feat), src.dtype),
        scratch_types=[
            pltpu.VMEM((per_w,), jnp.int32),
            pltpu.VMEM((per_w, feat), src.dtype),
            pltpu.SemaphoreType.DMA,
        ],
    )
    def k(src_hbm, idx_hbm, out_hbm, idx_v, rows_v, sem):
        wid = lax.axis_index("s") * mesh.num_cores + lax.axis_index("c")
        base = wid * per_w
        pltpu.sync_copy(idx_hbm.at[pl.ds(base, per_w)], idx_v)
        if scatter:
            pltpu.sync_copy(src_hbm.at[pl.ds(base, per_w)], rows_v)
            pltpu.async_copy(rows_v, out_hbm.at[idx_v], sem).wait()
        else:
            pltpu.async_copy(src_hbm.at[idx_v], rows_v, sem).wait()
            pltpu.sync_copy(rows_v, out_hbm.at[pl.ds(base, per_w)])

    return k(src, idx)


def _mm_body(meta_ref, x_ref, w_ref, g_ref, o_ref):
    w = pl.program_id(0)
    e_mask = meta_ref[2, w]
    first = meta_ref[3, w]
    xm = jnp.where(g_ref[...] == e_mask, x_ref[...], 0.0)
    part = lax.dot_general(
        xm, w_ref[0],
        dimension_numbers=(((1,), (1,)), ((), ())),
        preferred_element_type=jnp.float32,
    )

    @pl.when(first == 1)
    def _():
        o_ref[...] = part

    @pl.when(first == 0)
    def _():
        o_ref[...] += part


def _grouped_matmul(x_sorted, weight, sorted_gate, meta):
    grid_spec = pltpu.PrefetchScalarGridSpec(
        num_scalar_prefetch=1,
        grid=(_NW,),
        in_specs=[
            pl.BlockSpec((_BT, _IN), lambda w, m: (m[0, w], 0)),
            pl.BlockSpec((1, _OUT, _IN), lambda w, m: (m[1, w], 0, 0)),
            pl.BlockSpec((_BT, 1), lambda w, m: (m[0, w], 0)),
        ],
        out_specs=pl.BlockSpec((_BT, _OUT), lambda w, m: (m[0, w], 0)),
    )
    return pl.pallas_call(
        _mm_body,
        grid_spec=grid_spec,
        out_shape=jax.ShapeDtypeStruct((_TOKENS, _OUT), jnp.float32),
        compiler_params=pltpu.CompilerParams(
            dimension_semantics=("arbitrary",),
        ),
    )(meta, x_sorted, weight, sorted_gate.reshape(_TOKENS, 1))


def _routing(gate):
    g = gate.astype(jnp.int32)
    t = g.shape[0]
    eids = jnp.arange(_NUM_EXPERT, dtype=jnp.int32)
    oh3 = (g.reshape(128, t // 128, 1) == eids).astype(jnp.int32)
    c1 = jnp.cumsum(oh3, axis=1)
    row_tot = c1[:, -1, :]
    c2 = jnp.cumsum(row_tot, axis=0)
    excl = (c1 - oh3) + (c2 - row_tot)[:, None, :]
    pos = jnp.sum(oh3 * excl, axis=2).reshape(t)
    counts = c2[-1]
    off = jnp.cumsum(counts) - counts
    seg = jnp.sum(oh3 * off[None, None, :], axis=2).reshape(t)
    dest = (pos + seg).astype(jnp.int32)
    slot = jnp.arange(t, dtype=jnp.int32)
    sorted_gate = (jnp.sum((slot[:, None] >= off[None, :]).astype(jnp.int32),
                           axis=1) - 1).astype(jnp.int32)
    sgb = sorted_gate.reshape(_NB, _BT)
    e_lo, e_hi = sgb[:, 0], sgb[:, -1]
    nitem = e_hi - e_lo + 1
    starts = jnp.cumsum(nitem) - nitem
    total = jnp.sum(nitem)
    warr = jnp.arange(_NW, dtype=jnp.int32)
    b_of = jnp.sum((warr[:, None] >= starts[None, :]).astype(jnp.int32),
                   axis=1) - 1
    e_w = e_lo[b_of] + warr - starts[b_of]
    e_mask = jnp.where(warr < total, e_w, -1)
    e_load = jnp.clip(e_w, 0, _NUM_EXPERT - 1)
    firsts = (warr == starts[b_of]).astype(jnp.int32)
    meta = jnp.stack([b_of, e_load, e_mask, firsts]).astype(jnp.int32)
    return dest, sorted_gate, meta


def kernel(inp, gate, weight):
    dest, sorted_gate, meta = _routing(gate)
    x_sorted = _sc_permute(inp, dest, scatter=True)
    y_sorted = _grouped_matmul(x_sorted, weight, sorted_gate, meta)
    return _sc_permute(y_sorted, dest, scatter=False)


# speedup vs baseline: 1.4165x; 1.0536x over previous
import functools

import jax
import jax.numpy as jnp
from jax import lax
from jax.experimental import pallas as pl
from jax.experimental.pallas import tpu as pltpu
from jax.experimental.pallas import tpu_sc as plsc

_NUM_EXPERT = 8
_IN = 1024
_OUT = 1024
_TOKENS = 2048
_BT = 512
_NB = _TOKENS // _BT
_NW = _NB + _NUM_EXPERT - 1


def _sc_permute(src, idx, scatter):
    rows, feat = src.shape
    mesh = plsc.VectorSubcoreMesh(core_axis_name="c", subcore_axis_name="s")
    nworker = mesh.num_cores * mesh.num_subcores
    per_w = rows // nworker

    @functools.partial(
        pl.kernel,
        mesh=mesh,
        out_type=jax.ShapeDtypeStruct((rows, feat), src.dtype),
        scratch_types=[
            pltpu.VMEM((per_w,), jnp.int32),
            pltpu.VMEM((per_w, feat), src.dtype),
            pltpu.SemaphoreType.DMA,
        ],
    )
    def k(src_hbm, idx_hbm, out_hbm, idx_v, rows_v, sem):
        wid = lax.axis_index("s") * mesh.num_cores + lax.axis_index("c")
        base = wid * per_w
        pltpu.sync_copy(idx_hbm.at[pl.ds(base, per_w)], idx_v)
        if scatter:
            pltpu.sync_copy(src_hbm.at[pl.ds(base, per_w)], rows_v)
            pltpu.async_copy(rows_v, out_hbm.at[idx_v], sem).wait()
        else:
            pltpu.async_copy(src_hbm.at[idx_v], rows_v, sem).wait()
            pltpu.sync_copy(rows_v, out_hbm.at[pl.ds(base, per_w)])

    return k(src, idx)


def _mm_body(meta_ref, x_ref, w_ref, g_ref, o_ref):
    w = pl.program_id(0)
    e_mask = meta_ref[2, w]
    first = meta_ref[3, w]
    xm = jnp.where(g_ref[...] == e_mask, x_ref[...], 0.0)
    part = lax.dot_general(
        xm, w_ref[0],
        dimension_numbers=(((1,), (1,)), ((), ())),
        preferred_element_type=jnp.float32,
    )

    @pl.when(first == 1)
    def _():
        o_ref[...] = part

    @pl.when(first == 0)
    def _():
        o_ref[...] += part


def _grouped_matmul(x_sorted, weight, sorted_gate, meta):
    grid_spec = pltpu.PrefetchScalarGridSpec(
        num_scalar_prefetch=1,
        grid=(_NW,),
        in_specs=[
            pl.BlockSpec((_BT, _IN), lambda w, m: (m[0, w], 0)),
            pl.BlockSpec((1, _OUT, _IN), lambda w, m: (m[1, w], 0, 0)),
            pl.BlockSpec((_BT, 1), lambda w, m: (m[0, w], 0)),
        ],
        out_specs=pl.BlockSpec((_BT, _OUT), lambda w, m: (m[0, w], 0)),
    )
    return pl.pallas_call(
        _mm_body,
        grid_spec=grid_spec,
        out_shape=jax.ShapeDtypeStruct((_TOKENS, _OUT), jnp.float32),
        compiler_params=pltpu.CompilerParams(
            dimension_semantics=("arbitrary",),
        ),
    )(meta, x_sorted, weight, sorted_gate.reshape(_TOKENS, 1))


def _routing(gate):
    g = gate.astype(jnp.int32)
    t = g.shape[0]
    eids = jnp.arange(_NUM_EXPERT, dtype=jnp.int32)
    oh3 = (g.reshape(128, t // 128, 1) == eids).astype(jnp.int32)
    c1 = jnp.cumsum(oh3, axis=1)
    row_tot = c1[:, -1, :]
    c2 = jnp.cumsum(row_tot, axis=0)
    excl = (c1 - oh3) + (c2 - row_tot)[:, None, :]
    pos = jnp.sum(oh3 * excl, axis=2).reshape(t)
    counts = c2[-1]
    off = jnp.cumsum(counts) - counts
    seg = jnp.sum(oh3 * off[None, None, :], axis=2).reshape(t)
    dest = (pos + seg).astype(jnp.int32)
    slot = jnp.arange(t, dtype=jnp.int32)
    sorted_gate = (jnp.sum((slot[:, None] >= off[None, :]).astype(jnp.int32),
                           axis=1) - 1).astype(jnp.int32)
    sgb = sorted_gate.reshape(_NB, _BT)
    e_lo, e_hi = sgb[:, 0], sgb[:, -1]
    nitem = e_hi - e_lo + 1
    starts = jnp.cumsum(nitem) - nitem
    total = jnp.sum(nitem)
    warr = jnp.arange(_NW, dtype=jnp.int32)
    b_of = jnp.sum((warr[:, None] >= starts[None, :]).astype(jnp.int32),
                   axis=1) - 1
    e_w = e_lo[b_of] + warr - starts[b_of]
    e_mask = jnp.where(warr < total, e_w, -1)
    e_load = jnp.clip(e_w, 0, _NUM_EXPERT - 1)
    firsts = (warr == starts[b_of]).astype(jnp.int32)
    meta = jnp.stack([b_of, e_load, e_mask, firsts]).astype(jnp.int32)
    return dest, sorted_gate, meta


def kernel(inp, gate, weight):
    dest, sorted_gate, meta = _routing(gate)
    x_sorted = _sc_permute(inp, dest, scatter=True)
    y_sorted = _grouped_matmul(x_sorted, weight, sorted_gate, meta)
    return _sc_permute(y_sorted, dest, scatter=False)


# speedup vs baseline: 1.4222x; 1.0041x over previous
import functools

import jax
import jax.numpy as jnp
from jax import lax
from jax.experimental import pallas as pl
from jax.experimental.pallas import tpu as pltpu
from jax.experimental.pallas import tpu_sc as plsc

_NUM_EXPERT = 8
_IN = 1024
_OUT = 1024
_TOKENS = 2048
_BT = 512
_NB = _TOKENS // _BT
_NW = _NB + _NUM_EXPERT - 1


def _sc_permute(src, idx, scatter):
    rows, feat = src.shape
    mesh = plsc.VectorSubcoreMesh(core_axis_name="c", subcore_axis_name="s")
    nworker = mesh.num_cores * mesh.num_subcores
    per_w = rows // nworker

    @functools.partial(
        pl.kernel,
        mesh=mesh,
        out_type=jax.ShapeDtypeStruct((rows, feat), src.dtype),
        scratch_types=[
            pltpu.VMEM((per_w,), jnp.int32),
            pltpu.VMEM((per_w, feat), src.dtype),
            pltpu.SemaphoreType.DMA,
        ],
    )
    def k(src_hbm, idx_hbm, out_hbm, idx_v, rows_v, sem):
        wid = lax.axis_index("s") * mesh.num_cores + lax.axis_index("c")
        base = wid * per_w
        pltpu.sync_copy(idx_hbm.at[pl.ds(base, per_w)], idx_v)
        if scatter:
            pltpu.sync_copy(src_hbm.at[pl.ds(base, per_w)], rows_v)
            pltpu.async_copy(rows_v, out_hbm.at[idx_v], sem).wait()
        else:
            pltpu.async_copy(src_hbm.at[idx_v], rows_v, sem).wait()
            pltpu.sync_copy(rows_v, out_hbm.at[pl.ds(base, per_w)])

    return k(src, idx)


def _mm_body(meta_ref, x_ref, w_ref, o_ref):
    w = pl.program_id(0)
    first = meta_ref[2, w]
    lo = meta_ref[3, w]
    hi = meta_ref[4, w]
    rows = meta_ref[0, w] * _BT + lax.broadcasted_iota(
        jnp.int32, (_BT, 1), 0)
    mask = (rows >= lo) & (rows < hi)
    xm = jnp.where(mask, x_ref[...], 0.0)
    part = lax.dot_general(
        xm, w_ref[0],
        dimension_numbers=(((1,), (1,)), ((), ())),
        preferred_element_type=jnp.float32,
    )

    @pl.when(first == 1)
    def _():
        o_ref[...] = part

    @pl.when(first == 0)
    def _():
        o_ref[...] += part


def _grouped_matmul(x_sorted, weight, meta):
    grid_spec = pltpu.PrefetchScalarGridSpec(
        num_scalar_prefetch=1,
        grid=(_NW,),
        in_specs=[
            pl.BlockSpec((_BT, _IN), lambda w, m: (m[0, w], 0)),
            pl.BlockSpec((1, _OUT, _IN), lambda w, m: (m[1, w], 0, 0)),
        ],
        out_specs=pl.BlockSpec((_BT, _OUT), lambda w, m: (m[0, w], 0)),
    )
    return pl.pallas_call(
        _mm_body,
        grid_spec=grid_spec,
        out_shape=jax.ShapeDtypeStruct((_TOKENS, _OUT), jnp.float32),
        compiler_params=pltpu.CompilerParams(
            dimension_semantics=("arbitrary",),
        ),
    )(meta, x_sorted, weight)


def _routing(gate):
    g = gate.astype(jnp.int32)
    t = g.shape[0]
    eids = jnp.arange(_NUM_EXPERT, dtype=jnp.int32)
    oh3 = (g.reshape(128, t // 128, 1) == eids).astype(jnp.int32)
    c1 = jnp.cumsum(oh3, axis=1)
    row_tot = c1[:, -1, :]
    c2 = jnp.cumsum(row_tot, axis=0)
    excl = (c1 - oh3) + (c2 - row_tot)[:, None, :]
    pos = jnp.sum(oh3 * excl, axis=2).reshape(t)
    counts = c2[-1]
    off_end = jnp.cumsum(counts)
    off = off_end - counts
    seg = jnp.sum(oh3 * off[None, None, :], axis=2).reshape(t)
    dest = (pos + seg).astype(jnp.int32)
    blk_lo = jnp.arange(_NB, dtype=jnp.int32) * _BT
    e_lo = jnp.sum((off[None, :] <= blk_lo[:, None]).astype(jnp.int32),
                   axis=1) - 1
    e_hi = jnp.sum((off[None, :] <= blk_lo[:, None] + (_BT - 1)
                    ).astype(jnp.int32), axis=1) - 1
    nitem = e_hi - e_lo + 1
    starts = jnp.cumsum(nitem) - nitem
    total = jnp.sum(nitem)
    warr = jnp.arange(_NW, dtype=jnp.int32)
    b_of = jnp.sum((warr[:, None] >= starts[None, :]).astype(jnp.int32),
                   axis=1) - 1
    e_w = e_lo[b_of] + warr - starts[b_of]
    valid = warr < total
    e_load = jnp.clip(e_w, 0, _NUM_EXPERT - 1)
    firsts = (warr == starts[b_of]).astype(jnp.int32)
    seg_lo = jnp.where(valid, off[e_load], 0)
    seg_hi = jnp.where(valid, off_end[e_load], 0)
    meta = jnp.stack([b_of, e_load, firsts, seg_lo, seg_hi]
                     ).astype(jnp.int32)
    return dest, meta


def kernel(inp, gate, weight):
    dest, meta = _routing(gate)
    x_sorted = _sc_permute(inp, dest, scatter=True)
    y_sorted = _grouped_matmul(x_sorted, weight, meta)
    return _sc_permute(y_sorted, dest, scatter=False)
